# Initial kernel scaffold; baseline (speedup 1.0000x reference)
import jax
import jax.numpy as jnp
from jax.experimental import pallas as pl


def kernel(dst_ids, times, last_update):
    raise NotImplementedError("write your pallas kernel here")


# speedup vs baseline: 395.0070x; 395.0070x over previous
import functools

import jax
import jax.numpy as jnp
from jax import lax
from jax.experimental import pallas as pl
from jax.experimental.pallas import tpu as pltpu
from jax.experimental.pallas import tpu_sc as plsc

NC = 2
NS = 16
NW = NC * NS

NN = 100000
NN_PAD = 100352
ROWS = 784
NE = 6400000
EPT = NE // NW

C1 = 10000
NCH1 = EPT // C1

C3 = 2000
NCH3 = EPT // C3

NPS = NN_PAD // NS
APS = NPS // 4
NHALF = 4
IDXT = 112
NT_A = APS // IDXT

_mesh = plsc.VectorSubcoreMesh(core_axis_name="c", subcore_axis_name="s")
_sc_params = pltpu.CompilerParams(needs_layout_passes=False)


@functools.partial(
    pl.kernel,
    out_type=jax.ShapeDtypeStruct((NW, NN_PAD), jnp.int32),
    mesh=_mesh,
    scratch_types=[
        pltpu.VMEM((NN_PAD,), jnp.int32),
        pltpu.VMEM((C1,), jnp.int32),
        pltpu.VMEM((C1,), jnp.int32),
        pltpu.SemaphoreType.DMA,
        pltpu.SemaphoreType.DMA,
    ],
    compiler_params=_sc_params,
)
def _presence(dst_hbm, out_hbm, map_v, dst_b0, dst_b1, sem0, sem1):
    wid = lax.axis_index("s") * NC + lax.axis_index("c")
    base = wid * EPT
    sems = (sem0, sem1)
    bufs = (dst_b0, dst_b1)

    def zero_body(i, carry):
        for u in range(8):
            map_v[pl.ds((i * 8 + u) * 16, 16)] = jnp.zeros((16,), jnp.int32)
        return carry

    lax.fori_loop(0, NN_PAD // 128, zero_body, 0)

    ones = jnp.ones((16,), jnp.int32)
    pltpu.async_copy(dst_hbm.at[pl.ds(base, C1)], bufs[0], sems[0])
    pltpu.async_copy(dst_hbm.at[pl.ds(base + C1, C1)], bufs[1], sems[1])

    @pl.loop(0, NCH1, step=2)
    def _chunks(j0):
        for b in range(2):
            jj = j0 + b
            pltpu.make_async_copy(
                dst_hbm.at[pl.ds(base + jj * C1, C1)], bufs[b], sems[b]).wait()

            def sc_body(k, carry, _ref=bufs[b]):
                for u in range(5):
                    idx = _ref[pl.ds((k * 5 + u) * 16, 16)]
                    plsc.store_scatter(map_v, [idx], ones)
                return carry

            lax.fori_loop(0, C1 // 80, sc_body, 0)

            @pl.when(jj + 2 < NCH1)
            def _prefetch(_b=b, _jj=jj):
                pltpu.async_copy(
                    dst_hbm.at[pl.ds(base + (_jj + 2) * C1, C1)],
                    bufs[_b], sems[_b])

    pltpu.sync_copy(map_v, out_hbm.at[wid])


def _rank_body(maps_ref, rank_ref):
    acc = maps_ref[0]
    for i in range(1, NW):
        acc = acc | maps_ref[i]
    present = acc.astype(jnp.float32)

    r = lax.broadcasted_iota(jnp.int32, (128, 128), 0)
    c = lax.broadcasted_iota(jnp.int32, (128, 128), 1)
    upper = (r <= c).astype(jnp.float32)
    incl = jnp.dot(present, upper, preferred_element_type=jnp.float32)

    tot = jnp.broadcast_to(incl[:, 127:128], (ROWS, 128))
    ri = lax.broadcasted_iota(jnp.int32, (ROWS, ROWS), 0)
    ci = lax.broadcasted_iota(jnp.int32, (ROWS, ROWS), 1)
    strict_lower = (ci < ri).astype(jnp.float32)
    off = jnp.dot(strict_lower, tot, preferred_element_type=jnp.float32)

    rank_ref[...] = (incl + off - present).astype(jnp.int32)


_rank_tc = pl.pallas_call(
    _rank_body,
    out_shape=jax.ShapeDtypeStruct((ROWS, 128), jnp.int32),
)


@functools.partial(
    pl.kernel,
    out_type=jax.ShapeDtypeStruct((NE,), jnp.float32),
    mesh=_mesh,
    scratch_types=[
        pltpu.VMEM((NN_PAD,), jnp.int32),
        pltpu.VMEM((APS,), jnp.int32),
        pltpu.VMEM((APS,), jnp.int32),
        pltpu.VMEM((C3,), jnp.int32),
        pltpu.VMEM((C3,), jnp.int32),
        pltpu.VMEM((C3,), jnp.float32),
        pltpu.VMEM((C3,), jnp.float32),
        pltpu.VMEM((C3,), jnp.float32),
        pltpu.VMEM((C3,), jnp.float32),
        pltpu.VMEM_SHARED((NN_PAD,), jnp.int32),
        pltpu.SemaphoreType.DMA,
        pltpu.SemaphoreType.DMA,
        pltpu.SemaphoreType.DMA,
        pltpu.SemaphoreType.DMA,
        pltpu.SemaphoreType.DMA,
        pltpu.SemaphoreType.DMA,
        pltpu.SemaphoreType.DMA,
    ],
    compiler_params=_sc_params,
)
def _main(rank_hbm, dst_hbm, times_hbm, lu_hbm, out_hbm,
          tbl_v, rk_b, g1_b, dst_b0, dst_b1, t_b0, t_b1, o_b0, o_b1, tbl_sh,
          semA, semd0, semd1, semt0, semt1, semo0, semo1):
    cid = lax.axis_index("c")
    sid = lax.axis_index("s")
    wid = sid * NC + cid

    nbase = sid * NPS
    for half in range(NHALF):
        off = nbase + half * APS
        pltpu.sync_copy(rank_hbm.at[pl.ds(off, APS)], rk_b)
        hs = [pltpu.async_copy(
                  dst_hbm.at[rk_b.at[pl.ds(q * IDXT, IDXT)]],
                  g1_b.at[pl.ds(q * IDXT, IDXT)], semA)
              for q in range(NT_A)]
        for h in hs:
            h.wait()
        hs = [pltpu.async_copy(
                  lu_hbm.at[g1_b.at[pl.ds(q * IDXT, IDXT)]],
                  rk_b.at[pl.ds(q * IDXT, IDXT)], semA)
              for q in range(NT_A)]
        for h in hs:
            h.wait()

        def conv_body(k, carry):
            v = rk_b[pl.ds(k * 16, 16)]
            rk_b[pl.ds(k * 16, 16)] = plsc.bitcast(
                v.astype(jnp.float32), jnp.int32)
            return carry

        lax.fori_loop(0, APS // 16, conv_body, 0)
        pltpu.sync_copy(rk_b, tbl_sh.at[pl.ds(off, APS)])

    plsc.subcore_barrier()
    pltpu.sync_copy(tbl_sh, tbl_v)

    base = wid * EPT
    dst_b = (dst_b0, dst_b1)
    t_b = (t_b0, t_b1)
    o_b = (o_b0, o_b1)
    semd = (semd0, semd1)
    semt = (semt0, semt1)
    semo = (semo0, semo1)
    pltpu.async_copy(dst_hbm.at[pl.ds(base, C3)], dst_b[0], semd[0])
    pltpu.async_copy(times_hbm.at[pl.ds(base, C3)], t_b[0], semt[0])
    pltpu.async_copy(dst_hbm.at[pl.ds(base + C3, C3)], dst_b[1], semd[1])
    pltpu.async_copy(times_hbm.at[pl.ds(base + C3, C3)], t_b[1], semt[1])

    @pl.loop(0, NCH3, step=2)
    def _chunks(j0):
        for b in range(2):
            jj = j0 + b
            pltpu.make_async_copy(
                dst_hbm.at[pl.ds(base + jj * C3, C3)], dst_b[b], semd[b]).wait()
            pltpu.make_async_copy(
                times_hbm.at[pl.ds(base + jj * C3, C3)], t_b[b], semt[b]).wait()

            @pl.when(jj >= 2)
            def _drain_out(_b=b, _jj=jj):
                pltpu.make_async_copy(
                    o_b[_b], out_hbm.at[pl.ds(base + (_jj - 2) * C3, C3)],
                    semo[_b]).wait()

            def ev_body(k, carry, _d=dst_b[b], _t=t_b[b], _o=o_b[b]):
                for u in range(5):
                    s = pl.ds((k * 5 + u) * 16, 16)
                    idx = _d[s]
                    vals = plsc.bitcast(
                        plsc.load_gather(tbl_v, [idx]), jnp.float32)
                    _o[s] = vals - _t[s]
                return carry

            lax.fori_loop(0, C3 // 80, ev_body, 0)
            pltpu.async_copy(
                o_b[b], out_hbm.at[pl.ds(base + jj * C3, C3)], semo[b])

            @pl.when(jj + 2 < NCH3)
            def _prefetch(_b=b, _jj=jj):
                pltpu.async_copy(
                    dst_hbm.at[pl.ds(base + (_jj + 2) * C3, C3)],
                    dst_b[_b], semd[_b])
                pltpu.async_copy(
                    times_hbm.at[pl.ds(base + (_jj + 2) * C3, C3)],
                    t_b[_b], semt[_b])

    pltpu.make_async_copy(
        o_b[0], out_hbm.at[pl.ds(base + (NCH3 - 2) * C3, C3)], semo[0]).wait()
    pltpu.make_async_copy(
        o_b[1], out_hbm.at[pl.ds(base + (NCH3 - 1) * C3, C3)], semo[1]).wait()


def kernel(dst_ids, times, last_update):
    maps = _presence(dst_ids)
    rank = _rank_tc(maps.reshape(NW, ROWS, 128)).reshape(NN_PAD)
    return _main(rank, dst_ids, times, last_update)


# speedup vs baseline: 581.3319x; 1.4717x over previous
import functools

import jax
import jax.numpy as jnp
from jax import lax
from jax.experimental import pallas as pl
from jax.experimental.pallas import tpu as pltpu
from jax.experimental.pallas import tpu_sc as plsc

NC = 2
NS = 16
NW = NC * NS

NN = 100000
NN_PAD = 100352
ROWS = 784
NE = 6400000
EPT = NE // NW

C1 = 10000
NCH1 = EPT // C1

C3 = 2000
NCH3 = EPT // C3

NPS = NN_PAD // NS
APS = NPS // 4
NHALF = 4
IDXT = 112
NT_A = APS // IDXT

_mesh = plsc.VectorSubcoreMesh(core_axis_name="c", subcore_axis_name="s")
_sc_params = pltpu.CompilerParams(needs_layout_passes=False)


@functools.partial(
    pl.kernel,
    out_type=jax.ShapeDtypeStruct((NW, NN_PAD), jnp.int32),
    mesh=_mesh,
    scratch_types=[
        pltpu.VMEM((NN_PAD,), jnp.int32),
        pltpu.VMEM((C1,), jnp.int32),
        pltpu.VMEM((C1,), jnp.int32),
        pltpu.SemaphoreType.DMA,
        pltpu.SemaphoreType.DMA,
    ],
    compiler_params=_sc_params,
)
def _presence(dst_hbm, out_hbm, map_v, dst_b0, dst_b1, sem0, sem1):
    wid = lax.axis_index("s") * NC + lax.axis_index("c")
    base = wid * EPT
    sems = (sem0, sem1)
    bufs = (dst_b0, dst_b1)

    @plsc.parallel_loop(0, NN_PAD // 16, unroll=8)
    def _zero(i):
        map_v[pl.ds(i * 16, 16)] = jnp.zeros((16,), jnp.int32)

    ones = jnp.ones((16,), jnp.int32)
    pltpu.async_copy(dst_hbm.at[pl.ds(base, C1)], bufs[0], sems[0])
    pltpu.async_copy(dst_hbm.at[pl.ds(base + C1, C1)], bufs[1], sems[1])

    @pl.loop(0, NCH1, step=2)
    def _chunks(j0):
        for b in range(2):
            jj = j0 + b
            pltpu.make_async_copy(
                dst_hbm.at[pl.ds(base + jj * C1, C1)], bufs[b], sems[b]).wait()

            @plsc.parallel_loop(0, C1 // 16, unroll=8)
            def _scatter(k, _ref=bufs[b]):
                idx = _ref[pl.ds(k * 16, 16)]
                plsc.store_scatter(map_v, [idx], ones)

            @pl.when(jj + 2 < NCH1)
            def _prefetch(_b=b, _jj=jj):
                pltpu.async_copy(
                    dst_hbm.at[pl.ds(base + (_jj + 2) * C1, C1)],
                    bufs[_b], sems[_b])

    pltpu.sync_copy(map_v, out_hbm.at[wid])


def _rank_body(maps_ref, rank_ref):
    acc = maps_ref[0]
    for i in range(1, NW):
        acc = acc | maps_ref[i]
    present = acc.astype(jnp.float32)

    r = lax.broadcasted_iota(jnp.int32, (128, 128), 0)
    c = lax.broadcasted_iota(jnp.int32, (128, 128), 1)
    upper = (r <= c).astype(jnp.float32)
    incl = jnp.dot(present, upper, preferred_element_type=jnp.float32)

    tot = jnp.broadcast_to(incl[:, 127:128], (ROWS, 128))
    ri = lax.broadcasted_iota(jnp.int32, (ROWS, ROWS), 0)
    ci = lax.broadcasted_iota(jnp.int32, (ROWS, ROWS), 1)
    strict_lower = (ci < ri).astype(jnp.float32)
    off = jnp.dot(strict_lower, tot, preferred_element_type=jnp.float32)

    rank_ref[...] = (incl + off - present).astype(jnp.int32)


_rank_tc = pl.pallas_call(
    _rank_body,
    out_shape=jax.ShapeDtypeStruct((ROWS, 128), jnp.int32),
)


@functools.partial(
    pl.kernel,
    out_type=jax.ShapeDtypeStruct((NE,), jnp.float32),
    mesh=_mesh,
    scratch_types=[
        pltpu.VMEM((NN_PAD,), jnp.int32),
        pltpu.VMEM((APS,), jnp.int32),
        pltpu.VMEM((APS,), jnp.int32),
        pltpu.VMEM((C3,), jnp.int32),
        pltpu.VMEM((C3,), jnp.int32),
        pltpu.VMEM((C3,), jnp.float32),
        pltpu.VMEM((C3,), jnp.float32),
        pltpu.VMEM((C3,), jnp.float32),
        pltpu.VMEM((C3,), jnp.float32),
        pltpu.VMEM_SHARED((NN_PAD,), jnp.int32),
        pltpu.SemaphoreType.DMA,
        pltpu.SemaphoreType.DMA,
        pltpu.SemaphoreType.DMA,
        pltpu.SemaphoreType.DMA,
        pltpu.SemaphoreType.DMA,
        pltpu.SemaphoreType.DMA,
        pltpu.SemaphoreType.DMA,
    ],
    compiler_params=_sc_params,
)
def _main(rank_hbm, dst_hbm, times_hbm, lu_hbm, out_hbm,
          tbl_v, rk_b, g1_b, dst_b0, dst_b1, t_b0, t_b1, o_b0, o_b1, tbl_sh,
          semA, semd0, semd1, semt0, semt1, semo0, semo1):
    cid = lax.axis_index("c")
    sid = lax.axis_index("s")
    wid = sid * NC + cid

    nbase = sid * NPS
    for half in range(NHALF):
        off = nbase + half * APS
        pltpu.sync_copy(rank_hbm.at[pl.ds(off, APS)], rk_b)
        hs = [pltpu.async_copy(
                  dst_hbm.at[rk_b.at[pl.ds(q * IDXT, IDXT)]],
                  g1_b.at[pl.ds(q * IDXT, IDXT)], semA)
              for q in range(NT_A)]
        for h in hs:
            h.wait()
        hs = [pltpu.async_copy(
                  lu_hbm.at[g1_b.at[pl.ds(q * IDXT, IDXT)]],
                  rk_b.at[pl.ds(q * IDXT, IDXT)], semA)
              for q in range(NT_A)]
        for h in hs:
            h.wait()

        @plsc.parallel_loop(0, APS // 16, unroll=4)
        def _conv(k):
            v = rk_b[pl.ds(k * 16, 16)]
            rk_b[pl.ds(k * 16, 16)] = plsc.bitcast(
                v.astype(jnp.float32), jnp.int32)
        pltpu.sync_copy(rk_b, tbl_sh.at[pl.ds(off, APS)])

    plsc.subcore_barrier()
    pltpu.sync_copy(tbl_sh, tbl_v)

    base = wid * EPT
    dst_b = (dst_b0, dst_b1)
    t_b = (t_b0, t_b1)
    o_b = (o_b0, o_b1)
    semd = (semd0, semd1)
    semt = (semt0, semt1)
    semo = (semo0, semo1)
    pltpu.async_copy(dst_hbm.at[pl.ds(base, C3)], dst_b[0], semd[0])
    pltpu.async_copy(times_hbm.at[pl.ds(base, C3)], t_b[0], semt[0])
    pltpu.async_copy(dst_hbm.at[pl.ds(base + C3, C3)], dst_b[1], semd[1])
    pltpu.async_copy(times_hbm.at[pl.ds(base + C3, C3)], t_b[1], semt[1])

    @pl.loop(0, NCH3, step=2)
    def _chunks(j0):
        for b in range(2):
            jj = j0 + b
            pltpu.make_async_copy(
                dst_hbm.at[pl.ds(base + jj * C3, C3)], dst_b[b], semd[b]).wait()
            pltpu.make_async_copy(
                times_hbm.at[pl.ds(base + jj * C3, C3)], t_b[b], semt[b]).wait()

            @pl.when(jj >= 2)
            def _drain_out(_b=b, _jj=jj):
                pltpu.make_async_copy(
                    o_b[_b], out_hbm.at[pl.ds(base + (_jj - 2) * C3, C3)],
                    semo[_b]).wait()

            @plsc.parallel_loop(0, C3 // 16, unroll=8)
            def _ev(k, _d=dst_b[b], _t=t_b[b], _o=o_b[b]):
                s = pl.ds(k * 16, 16)
                idx = _d[s]
                vals = plsc.bitcast(
                    plsc.load_gather(tbl_v, [idx]), jnp.float32)
                _o[s] = vals - _t[s]
            pltpu.async_copy(
                o_b[b], out_hbm.at[pl.ds(base + jj * C3, C3)], semo[b])

            @pl.when(jj + 2 < NCH3)
            def _prefetch(_b=b, _jj=jj):
                pltpu.async_copy(
                    dst_hbm.at[pl.ds(base + (_jj + 2) * C3, C3)],
                    dst_b[_b], semd[_b])
                pltpu.async_copy(
                    times_hbm.at[pl.ds(base + (_jj + 2) * C3, C3)],
                    t_b[_b], semt[_b])

    pltpu.make_async_copy(
        o_b[0], out_hbm.at[pl.ds(base + (NCH3 - 2) * C3, C3)], semo[0]).wait()
    pltpu.make_async_copy(
        o_b[1], out_hbm.at[pl.ds(base + (NCH3 - 1) * C3, C3)], semo[1]).wait()


def kernel(dst_ids, times, last_update):
    maps = _presence(dst_ids)
    rank = _rank_tc(maps.reshape(NW, ROWS, 128)).reshape(NN_PAD)
    return _main(rank, dst_ids, times, last_update)


# speedup vs baseline: 663.4038x; 1.1412x over previous
import functools

import jax
import jax.numpy as jnp
from jax import lax
from jax.experimental import pallas as pl
from jax.experimental.pallas import tpu as pltpu
from jax.experimental.pallas import tpu_sc as plsc

NC = 2
NS = 16
NW = NC * NS

NN = 100000
NN_PAD = 100352
ROWS = 784
NE = 6400000
EPT = NE // NW

C1 = 10000
NCH1 = EPT // C1

C3 = 2000
NCH3 = EPT // C3

NPS = NN_PAD // NS
APS = NPS // 2
NHALF = 2
IDXT = 112
NT_A = APS // IDXT

_mesh = plsc.VectorSubcoreMesh(core_axis_name="c", subcore_axis_name="s")
_sc_params = pltpu.CompilerParams(needs_layout_passes=False)


@functools.partial(
    pl.kernel,
    out_type=jax.ShapeDtypeStruct((NW, ROWS, 128), jnp.int32),
    mesh=_mesh,
    scratch_types=[
        pltpu.VMEM((ROWS, 128), jnp.int32),
        pltpu.VMEM((C1,), jnp.int32),
        pltpu.VMEM((C1,), jnp.int32),
        pltpu.SemaphoreType.DMA,
        pltpu.SemaphoreType.DMA,
    ],
    compiler_params=_sc_params,
)
def _presence(dst_hbm, out_hbm, map_v, dst_b0, dst_b1, sem0, sem1):
    wid = lax.axis_index("s") * NC + lax.axis_index("c")
    base = wid * EPT
    sems = (sem0, sem1)
    bufs = (dst_b0, dst_b1)

    @plsc.parallel_loop(0, ROWS, unroll=8)
    def _zero(i):
        map_v[i, pl.ds(0, 16)] = jnp.zeros((16,), jnp.int32)
        map_v[i, pl.ds(16, 16)] = jnp.zeros((16,), jnp.int32)
        map_v[i, pl.ds(32, 16)] = jnp.zeros((16,), jnp.int32)
        map_v[i, pl.ds(48, 16)] = jnp.zeros((16,), jnp.int32)
        map_v[i, pl.ds(64, 16)] = jnp.zeros((16,), jnp.int32)
        map_v[i, pl.ds(80, 16)] = jnp.zeros((16,), jnp.int32)
        map_v[i, pl.ds(96, 16)] = jnp.zeros((16,), jnp.int32)
        map_v[i, pl.ds(112, 16)] = jnp.zeros((16,), jnp.int32)

    ones = jnp.ones((16,), jnp.int32)
    pltpu.async_copy(dst_hbm.at[pl.ds(base, C1)], bufs[0], sems[0])
    pltpu.async_copy(dst_hbm.at[pl.ds(base + C1, C1)], bufs[1], sems[1])

    @pl.loop(0, NCH1, step=2)
    def _chunks(j0):
        for b in range(2):
            jj = j0 + b
            pltpu.make_async_copy(
                dst_hbm.at[pl.ds(base + jj * C1, C1)], bufs[b], sems[b]).wait()

            @plsc.parallel_loop(0, C1 // 16, unroll=8)
            def _scatter(k, _ref=bufs[b]):
                idx = _ref[pl.ds(k * 16, 16)]
                row = lax.shift_right_logical(idx, 7)
                col = lax.bitwise_and(idx, jnp.int32(127))
                plsc.store_scatter(map_v, [row, col], ones)

            @pl.when(jj + 2 < NCH1)
            def _prefetch(_b=b, _jj=jj):
                pltpu.async_copy(
                    dst_hbm.at[pl.ds(base + (_jj + 2) * C1, C1)],
                    bufs[_b], sems[_b])

    pltpu.sync_copy(map_v, out_hbm.at[wid])


def _rank_body(maps_ref, rank_ref):
    acc = maps_ref[0]
    for i in range(1, NW):
        acc = acc | maps_ref[i]
    present = acc.astype(jnp.float32)

    r = lax.broadcasted_iota(jnp.int32, (128, 128), 0)
    c = lax.broadcasted_iota(jnp.int32, (128, 128), 1)
    upper = (r <= c).astype(jnp.float32)
    incl = jnp.dot(present, upper, preferred_element_type=jnp.float32)

    tot = jnp.broadcast_to(incl[:, 127:128], (ROWS, 128))
    ri = lax.broadcasted_iota(jnp.int32, (ROWS, ROWS), 0)
    ci = lax.broadcasted_iota(jnp.int32, (ROWS, ROWS), 1)
    strict_lower = (ci < ri).astype(jnp.float32)
    off = jnp.dot(strict_lower, tot, preferred_element_type=jnp.float32)

    rank_ref[...] = (incl + off - present).astype(jnp.int32)


_rank_tc = pl.pallas_call(
    _rank_body,
    out_shape=jax.ShapeDtypeStruct((ROWS, 128), jnp.int32),
)


@functools.partial(
    pl.kernel,
    out_type=jax.ShapeDtypeStruct((NE,), jnp.float32),
    mesh=_mesh,
    scratch_types=[
        pltpu.VMEM((NN_PAD,), jnp.int32),
        pltpu.VMEM((APS,), jnp.int32),
        pltpu.VMEM((APS,), jnp.int32),
        pltpu.VMEM((C3,), jnp.int32),
        pltpu.VMEM((C3,), jnp.int32),
        pltpu.VMEM((C3,), jnp.float32),
        pltpu.VMEM((C3,), jnp.float32),
        pltpu.VMEM((C3,), jnp.float32),
        pltpu.VMEM((C3,), jnp.float32),
        pltpu.VMEM_SHARED((NN_PAD,), jnp.int32),
        pltpu.SemaphoreType.DMA,
        pltpu.SemaphoreType.DMA,
        pltpu.SemaphoreType.DMA,
        pltpu.SemaphoreType.DMA,
        pltpu.SemaphoreType.DMA,
        pltpu.SemaphoreType.DMA,
        pltpu.SemaphoreType.DMA,
    ],
    compiler_params=_sc_params,
)
def _main(rank_hbm, dst_hbm, times_hbm, lu_hbm, out_hbm,
          tbl_v, rk_b, g1_b, dst_b0, dst_b1, t_b0, t_b1, o_b0, o_b1, tbl_sh,
          semA, semd0, semd1, semt0, semt1, semo0, semo1):
    cid = lax.axis_index("c")
    sid = lax.axis_index("s")
    wid = sid * NC + cid
    base = wid * EPT
    dst_b = (dst_b0, dst_b1)
    t_b = (t_b0, t_b1)
    o_b = (o_b0, o_b1)
    semd = (semd0, semd1)
    semt = (semt0, semt1)
    semo = (semo0, semo1)

    pltpu.async_copy(dst_hbm.at[pl.ds(base, C3)], dst_b[0], semd[0])
    pltpu.async_copy(times_hbm.at[pl.ds(base, C3)], t_b[0], semt[0])
    pltpu.async_copy(dst_hbm.at[pl.ds(base + C3, C3)], dst_b[1], semd[1])
    pltpu.async_copy(times_hbm.at[pl.ds(base + C3, C3)], t_b[1], semt[1])

    with jax.named_scope("phaseA"):
        nbase = sid * NPS
        for half in range(NHALF):
            off = nbase + half * APS
            pltpu.sync_copy(rank_hbm.at[pl.ds(off, APS)], rk_b)
            hs = [pltpu.async_copy(
                      dst_hbm.at[rk_b.at[pl.ds(q * IDXT, IDXT)]],
                      g1_b.at[pl.ds(q * IDXT, IDXT)], semA)
                  for q in range(NT_A)]
            for h in hs:
                h.wait()
            hs = [pltpu.async_copy(
                      lu_hbm.at[g1_b.at[pl.ds(q * IDXT, IDXT)]],
                      rk_b.at[pl.ds(q * IDXT, IDXT)], semA)
                  for q in range(NT_A)]
            for h in hs:
                h.wait()

            @plsc.parallel_loop(0, APS // 16, unroll=4)
            def _conv(k):
                v = rk_b[pl.ds(k * 16, 16)]
                rk_b[pl.ds(k * 16, 16)] = plsc.bitcast(
                    v.astype(jnp.float32), jnp.int32)
            pltpu.sync_copy(rk_b, tbl_sh.at[pl.ds(off, APS)])

    plsc.subcore_barrier()
    with jax.named_scope("broadcast"):
        pltpu.sync_copy(tbl_sh, tbl_v)


    @pl.loop(0, NCH3, step=2)
    def _chunks(j0):
        for b in range(2):
            jj = j0 + b
            pltpu.make_async_copy(
                dst_hbm.at[pl.ds(base + jj * C3, C3)], dst_b[b], semd[b]).wait()
            pltpu.make_async_copy(
                times_hbm.at[pl.ds(base + jj * C3, C3)], t_b[b], semt[b]).wait()

            @pl.when(jj >= 2)
            def _drain_out(_b=b, _jj=jj):
                pltpu.make_async_copy(
                    o_b[_b], out_hbm.at[pl.ds(base + (_jj - 2) * C3, C3)],
                    semo[_b]).wait()

            @plsc.parallel_loop(0, C3 // 16, unroll=8)
            def _ev(k, _d=dst_b[b], _t=t_b[b], _o=o_b[b]):
                s = pl.ds(k * 16, 16)
                idx = _d[s]
                vals = plsc.bitcast(
                    plsc.load_gather(tbl_v, [idx]), jnp.float32)
                _o[s] = vals - _t[s]
            pltpu.async_copy(
                o_b[b], out_hbm.at[pl.ds(base + jj * C3, C3)], semo[b])

            @pl.when(jj + 2 < NCH3)
            def _prefetch(_b=b, _jj=jj):
                pltpu.async_copy(
                    dst_hbm.at[pl.ds(base + (_jj + 2) * C3, C3)],
                    dst_b[_b], semd[_b])
                pltpu.async_copy(
                    times_hbm.at[pl.ds(base + (_jj + 2) * C3, C3)],
                    t_b[_b], semt[_b])

    pltpu.make_async_copy(
        o_b[0], out_hbm.at[pl.ds(base + (NCH3 - 2) * C3, C3)], semo[0]).wait()
    pltpu.make_async_copy(
        o_b[1], out_hbm.at[pl.ds(base + (NCH3 - 1) * C3, C3)], semo[1]).wait()


def kernel(dst_ids, times, last_update):
    maps = _presence(dst_ids)
    rank = _rank_tc(maps).reshape(NN_PAD)
    return _main(rank, dst_ids, times, last_update)


# speedup vs baseline: 664.2167x; 1.0012x over previous
import functools

import jax
import jax.numpy as jnp
from jax import lax
from jax.experimental import pallas as pl
from jax.experimental.pallas import tpu as pltpu
from jax.experimental.pallas import tpu_sc as plsc

NC = 2
NS = 16
NW = NC * NS

NN = 100000
NN_PAD = 100352
ROWS = 784
NE = 6400000
EPT = NE // NW

C1 = 5000
NCH1 = EPT // C1
PPT = NN_PAD // NW

C3 = 2000
NCH3 = EPT // C3

NPS = NN_PAD // NS
APS = NPS // 2
NHALF = 2
IDXT = 112
NT_A = APS // IDXT

_mesh = plsc.VectorSubcoreMesh(core_axis_name="c", subcore_axis_name="s")
_sc_params = pltpu.CompilerParams(needs_layout_passes=False)


@functools.partial(
    pl.kernel,
    out_type=(
        jax.ShapeDtypeStruct((NW, ROWS, 128), jnp.int32),
        jax.ShapeDtypeStruct((NN_PAD,), jnp.int32),
    ),
    mesh=_mesh,
    scratch_types=[
        pltpu.VMEM((ROWS, 128), jnp.int32),
        pltpu.VMEM((C1,), jnp.int32),
        pltpu.VMEM((C1,), jnp.int32),
        pltpu.VMEM((PPT,), jnp.int32),
        pltpu.VMEM((PPT,), jnp.int32),
        pltpu.SemaphoreType.DMA,
        pltpu.SemaphoreType.DMA,
        pltpu.SemaphoreType.DMA,
    ],
    compiler_params=_sc_params,
)
def _presence(dst_hbm, lu_hbm, out_hbm, mid_hbm,
              map_v, dst_b0, dst_b1, pfx_b, val_b, sem0, sem1, semM):
    wid = lax.axis_index("s") * NC + lax.axis_index("c")
    base = wid * EPT
    sems = (sem0, sem1)
    bufs = (dst_b0, dst_b1)

    pbase = wid * PPT
    pltpu.sync_copy(dst_hbm.at[pl.ds(pbase, PPT)], pfx_b)
    mid_hs = [pltpu.async_copy(
                  lu_hbm.at[pfx_b.at[pl.ds(q * IDXT, IDXT)]],
                  val_b.at[pl.ds(q * IDXT, IDXT)], semM)
              for q in range(PPT // IDXT)]

    @plsc.parallel_loop(0, ROWS, unroll=8)
    def _zero(i):
        map_v[i, pl.ds(0, 16)] = jnp.zeros((16,), jnp.int32)
        map_v[i, pl.ds(16, 16)] = jnp.zeros((16,), jnp.int32)
        map_v[i, pl.ds(32, 16)] = jnp.zeros((16,), jnp.int32)
        map_v[i, pl.ds(48, 16)] = jnp.zeros((16,), jnp.int32)
        map_v[i, pl.ds(64, 16)] = jnp.zeros((16,), jnp.int32)
        map_v[i, pl.ds(80, 16)] = jnp.zeros((16,), jnp.int32)
        map_v[i, pl.ds(96, 16)] = jnp.zeros((16,), jnp.int32)
        map_v[i, pl.ds(112, 16)] = jnp.zeros((16,), jnp.int32)

    ones = jnp.ones((16,), jnp.int32)
    pltpu.async_copy(dst_hbm.at[pl.ds(base, C1)], bufs[0], sems[0])
    pltpu.async_copy(dst_hbm.at[pl.ds(base + C1, C1)], bufs[1], sems[1])

    @pl.loop(0, NCH1, step=2)
    def _chunks(j0):
        for b in range(2):
            jj = j0 + b
            pltpu.make_async_copy(
                dst_hbm.at[pl.ds(base + jj * C1, C1)], bufs[b], sems[b]).wait()

            @plsc.parallel_loop(0, C1 // 16, unroll=8)
            def _scatter(k, _ref=bufs[b]):
                idx = _ref[pl.ds(k * 16, 16)]
                row = lax.shift_right_logical(idx, 7)
                col = lax.bitwise_and(idx, jnp.int32(127))
                plsc.store_scatter(map_v, [row, col], ones)

            @pl.when(jj + 2 < NCH1)
            def _prefetch(_b=b, _jj=jj):
                pltpu.async_copy(
                    dst_hbm.at[pl.ds(base + (_jj + 2) * C1, C1)],
                    bufs[_b], sems[_b])

    pltpu.sync_copy(map_v, out_hbm.at[wid])

    for h in mid_hs:
        h.wait()

    @plsc.parallel_loop(0, PPT // 16, unroll=4)
    def _conv(k):
        v = val_b[pl.ds(k * 16, 16)]
        val_b[pl.ds(k * 16, 16)] = plsc.bitcast(v.astype(jnp.float32),
                                                jnp.int32)

    pltpu.sync_copy(val_b, mid_hbm.at[pl.ds(pbase, PPT)])


def _rank_body(maps_ref, rank_ref):
    acc = maps_ref[0]
    for i in range(1, NW):
        acc = acc | maps_ref[i]
    present = acc.astype(jnp.float32)

    r = lax.broadcasted_iota(jnp.int32, (128, 128), 0)
    c = lax.broadcasted_iota(jnp.int32, (128, 128), 1)
    upper = (r <= c).astype(jnp.float32)
    incl = jnp.dot(present, upper, preferred_element_type=jnp.float32)

    tot = jnp.broadcast_to(incl[:, 127:128], (ROWS, 128))
    ri = lax.broadcasted_iota(jnp.int32, (ROWS, ROWS), 0)
    ci = lax.broadcasted_iota(jnp.int32, (ROWS, ROWS), 1)
    strict_lower = (ci < ri).astype(jnp.float32)
    off = jnp.dot(strict_lower, tot, preferred_element_type=jnp.float32)

    rank_ref[...] = (incl + off - present).astype(jnp.int32)


_rank_tc = pl.pallas_call(
    _rank_body,
    out_shape=jax.ShapeDtypeStruct((ROWS, 128), jnp.int32),
)


@functools.partial(
    pl.kernel,
    out_type=jax.ShapeDtypeStruct((NE,), jnp.float32),
    mesh=_mesh,
    scratch_types=[
        pltpu.VMEM((NN_PAD,), jnp.int32),
        pltpu.VMEM((APS,), jnp.int32),
        pltpu.VMEM((APS,), jnp.int32),
        pltpu.VMEM((C3,), jnp.int32),
        pltpu.VMEM((C3,), jnp.int32),
        pltpu.VMEM((C3,), jnp.float32),
        pltpu.VMEM((C3,), jnp.float32),
        pltpu.VMEM((C3,), jnp.float32),
        pltpu.VMEM((C3,), jnp.float32),
        pltpu.VMEM_SHARED((NN_PAD,), jnp.int32),
        pltpu.SemaphoreType.DMA,
        pltpu.SemaphoreType.DMA,
        pltpu.SemaphoreType.DMA,
        pltpu.SemaphoreType.DMA,
        pltpu.SemaphoreType.DMA,
        pltpu.SemaphoreType.DMA,
        pltpu.SemaphoreType.DMA,
    ],
    compiler_params=_sc_params,
)
def _main(rank_hbm, dst_hbm, times_hbm, mid_hbm, out_hbm,
          tbl_v, rk_b, g1_b, dst_b0, dst_b1, t_b0, t_b1, o_b0, o_b1, tbl_sh,
          semA, semd0, semd1, semt0, semt1, semo0, semo1):
    cid = lax.axis_index("c")
    sid = lax.axis_index("s")
    wid = sid * NC + cid
    base = wid * EPT
    dst_b = (dst_b0, dst_b1)
    t_b = (t_b0, t_b1)
    o_b = (o_b0, o_b1)
    semd = (semd0, semd1)
    semt = (semt0, semt1)
    semo = (semo0, semo1)

    pltpu.async_copy(dst_hbm.at[pl.ds(base, C3)], dst_b[0], semd[0])
    pltpu.async_copy(times_hbm.at[pl.ds(base, C3)], t_b[0], semt[0])
    pltpu.async_copy(dst_hbm.at[pl.ds(base + C3, C3)], dst_b[1], semd[1])
    pltpu.async_copy(times_hbm.at[pl.ds(base + C3, C3)], t_b[1], semt[1])

    with jax.named_scope("phaseA"):
        nbase = sid * NPS
        pltpu.sync_copy(rank_hbm.at[pl.ds(nbase, APS)], rk_b)
        for half in range(NHALF):
            off = nbase + half * APS
            hs = [pltpu.async_copy(
                      mid_hbm.at[rk_b.at[pl.ds(q * IDXT, IDXT)]],
                      g1_b.at[pl.ds(q * IDXT, IDXT)], semA)
                  for q in range(NT_A)]
            for h in hs:
                h.wait()
            if half + 1 < NHALF:
                pltpu.sync_copy(
                    rank_hbm.at[pl.ds(off + APS, APS)], rk_b)
            pltpu.sync_copy(g1_b, tbl_sh.at[pl.ds(off, APS)])

    plsc.subcore_barrier()
    with jax.named_scope("broadcast"):
        pltpu.sync_copy(tbl_sh, tbl_v)


    @pl.loop(0, NCH3, step=2)
    def _chunks(j0):
        for b in range(2):
            jj = j0 + b
            pltpu.make_async_copy(
                dst_hbm.at[pl.ds(base + jj * C3, C3)], dst_b[b], semd[b]).wait()
            pltpu.make_async_copy(
                times_hbm.at[pl.ds(base + jj * C3, C3)], t_b[b], semt[b]).wait()

            @pl.when(jj >= 2)
            def _drain_out(_b=b, _jj=jj):
                pltpu.make_async_copy(
                    o_b[_b], out_hbm.at[pl.ds(base + (_jj - 2) * C3, C3)],
                    semo[_b]).wait()

            @plsc.parallel_loop(0, C3 // 16, unroll=8)
            def _ev(k, _d=dst_b[b], _t=t_b[b], _o=o_b[b]):
                s = pl.ds(k * 16, 16)
                idx = _d[s]
                vals = plsc.bitcast(
                    plsc.load_gather(tbl_v, [idx]), jnp.float32)
                _o[s] = vals - _t[s]
            pltpu.async_copy(
                o_b[b], out_hbm.at[pl.ds(base + jj * C3, C3)], semo[b])

            @pl.when(jj + 2 < NCH3)
            def _prefetch(_b=b, _jj=jj):
                pltpu.async_copy(
                    dst_hbm.at[pl.ds(base + (_jj + 2) * C3, C3)],
                    dst_b[_b], semd[_b])
                pltpu.async_copy(
                    times_hbm.at[pl.ds(base + (_jj + 2) * C3, C3)],
                    t_b[_b], semt[_b])

    pltpu.make_async_copy(
        o_b[0], out_hbm.at[pl.ds(base + (NCH3 - 2) * C3, C3)], semo[0]).wait()
    pltpu.make_async_copy(
        o_b[1], out_hbm.at[pl.ds(base + (NCH3 - 1) * C3, C3)], semo[1]).wait()


def kernel(dst_ids, times, last_update):
    maps, mid = _presence(dst_ids, last_update)
    rank = _rank_tc(maps).reshape(NN_PAD)
    return _main(rank, dst_ids, times, mid)


# speedup vs baseline: 921.2858x; 1.3870x over previous
import functools

import jax
import jax.numpy as jnp
from jax import lax
from jax.experimental import pallas as pl
from jax.experimental.pallas import tpu as pltpu
from jax.experimental.pallas import tpu_sc as plsc

NC = 2
NS = 16
NW = NC * NS

NN = 100000
NN_PAD = 100352
ROWS = 784
NE = 6400000
EPT = NE // NW

C1 = 10000
NCH1 = EPT // C1
PPT = NN_PAD // NW

C3 = 4000
NCH3 = EPT // C3

NPS = NN_PAD // NS
APS = NPS // 2
NHALF = 2
WIN = 3200
IDXT = 112

_mesh = plsc.VectorSubcoreMesh(core_axis_name="c", subcore_axis_name="s")
_sc_params = pltpu.CompilerParams(needs_layout_passes=False)


@functools.partial(
    pl.kernel,
    out_type=(
        jax.ShapeDtypeStruct((NW, ROWS, 128), jnp.int32),
        jax.ShapeDtypeStruct((NN_PAD,), jnp.int32),
    ),
    mesh=_mesh,
    scratch_types=[
        pltpu.VMEM((ROWS, 128), jnp.int32),
        pltpu.VMEM((C1,), jnp.int32),
        pltpu.VMEM((C1,), jnp.int32),
        pltpu.VMEM((PPT,), jnp.int32),
        pltpu.VMEM((PPT,), jnp.int32),
        pltpu.SemaphoreType.DMA,
        pltpu.SemaphoreType.DMA,
        pltpu.SemaphoreType.DMA,
    ],
    compiler_params=_sc_params,
)
def _presence(dst_hbm, lu_hbm, out_hbm, mid_hbm,
              map_v, dst_b0, dst_b1, pfx_b, val_b, sem0, sem1, semM):
    wid = lax.axis_index("s") * NC + lax.axis_index("c")
    base = wid * EPT
    sems = (sem0, sem1)
    bufs = (dst_b0, dst_b1)

    pltpu.async_copy(dst_hbm.at[pl.ds(base, C1)], bufs[0], sems[0])
    pltpu.async_copy(dst_hbm.at[pl.ds(base + C1, C1)], bufs[1], sems[1])

    pbase = wid * PPT
    pltpu.sync_copy(dst_hbm.at[pl.ds(pbase, PPT)], pfx_b)
    mid_hs = [pltpu.async_copy(
                  lu_hbm.at[pfx_b.at[pl.ds(q * IDXT, IDXT)]],
                  val_b.at[pl.ds(q * IDXT, IDXT)], semM)
              for q in range(PPT // IDXT)]

    @plsc.parallel_loop(0, ROWS, unroll=8)
    def _zero(i):
        map_v[i, pl.ds(0, 16)] = jnp.zeros((16,), jnp.int32)
        map_v[i, pl.ds(16, 16)] = jnp.zeros((16,), jnp.int32)
        map_v[i, pl.ds(32, 16)] = jnp.zeros((16,), jnp.int32)
        map_v[i, pl.ds(48, 16)] = jnp.zeros((16,), jnp.int32)
        map_v[i, pl.ds(64, 16)] = jnp.zeros((16,), jnp.int32)
        map_v[i, pl.ds(80, 16)] = jnp.zeros((16,), jnp.int32)
        map_v[i, pl.ds(96, 16)] = jnp.zeros((16,), jnp.int32)
        map_v[i, pl.ds(112, 16)] = jnp.zeros((16,), jnp.int32)

    ones = jnp.ones((16,), jnp.int32)

    @pl.loop(0, NCH1, step=2)
    def _chunks(j0):
        for b in range(2):
            jj = j0 + b
            pltpu.make_async_copy(
                dst_hbm.at[pl.ds(base + jj * C1, C1)], bufs[b], sems[b]).wait()

            @plsc.parallel_loop(0, C1 // 16, unroll=5)
            def _scatter(k, _ref=bufs[b]):
                idx = _ref[pl.ds(k * 16, 16)]
                row = lax.shift_right_logical(idx, 7)
                col = lax.bitwise_and(idx, jnp.int32(127))
                plsc.store_scatter(map_v, [row, col], ones)

            @pl.when(jj + 2 < NCH1)
            def _prefetch(_b=b, _jj=jj):
                pltpu.async_copy(
                    dst_hbm.at[pl.ds(base + (_jj + 2) * C1, C1)],
                    bufs[_b], sems[_b])

    map_h = pltpu.async_copy(map_v, out_hbm.at[wid], sem0)

    for h in mid_hs:
        h.wait()

    @plsc.parallel_loop(0, PPT // 16, unroll=4)
    def _conv(k):
        v = val_b[pl.ds(k * 16, 16)]
        val_b[pl.ds(k * 16, 16)] = plsc.bitcast(v.astype(jnp.float32),
                                                jnp.int32)

    pltpu.sync_copy(val_b, mid_hbm.at[pl.ds(pbase, PPT)])
    map_h.wait()


def _rank_body(maps_ref, rank_ref):
    acc = maps_ref[0]
    for i in range(1, NW):
        acc = acc | maps_ref[i]
    present = acc.astype(jnp.float32)

    r = lax.broadcasted_iota(jnp.int32, (128, 128), 0)
    c = lax.broadcasted_iota(jnp.int32, (128, 128), 1)
    upper = (r <= c).astype(jnp.float32)
    incl = jnp.dot(present, upper, preferred_element_type=jnp.float32)

    tot = jnp.broadcast_to(incl[:, 127:128], (ROWS, 128))
    ri = lax.broadcasted_iota(jnp.int32, (ROWS, ROWS), 0)
    ci = lax.broadcasted_iota(jnp.int32, (ROWS, ROWS), 1)
    strict_lower = (ci < ri).astype(jnp.float32)
    off = jnp.dot(strict_lower, tot, preferred_element_type=jnp.float32)

    rank_ref[...] = (incl + off - present).astype(jnp.int32)


_rank_tc = pl.pallas_call(
    _rank_body,
    out_shape=jax.ShapeDtypeStruct((ROWS, 128), jnp.int32),
)


@functools.partial(
    pl.kernel,
    out_type=jax.ShapeDtypeStruct((NE,), jnp.float32),
    mesh=_mesh,
    scratch_types=[
        pltpu.VMEM((NN,), jnp.float32),
        pltpu.VMEM((C3,), jnp.int32),
        pltpu.VMEM((C3,), jnp.int32),
        pltpu.VMEM((C3,), jnp.float32),
        pltpu.VMEM((C3,), jnp.float32),
        pltpu.VMEM((C3,), jnp.float32),
        pltpu.VMEM((C3,), jnp.float32),
        pltpu.VMEM_SHARED((NN_PAD,), jnp.float32),
        pltpu.SemaphoreType.DMA,
        pltpu.SemaphoreType.DMA,
        pltpu.SemaphoreType.DMA,
        pltpu.SemaphoreType.DMA,
        pltpu.SemaphoreType.DMA,
        pltpu.SemaphoreType.DMA,
    ],
    compiler_params=_sc_params,
)
def _main(rank_hbm, dst_hbm, times_hbm, mid_hbm, out_hbm,
          tbl_v, dst_b0, dst_b1, t_b0, t_b1, o_b0, o_b1, tbl_sh,
          semd0, semd1, semt0, semt1, semo0, semo1):
    cid = lax.axis_index("c")
    sid = lax.axis_index("s")
    wid = sid * NC + cid
    base = wid * EPT
    dst_b = (dst_b0, dst_b1)
    t_b = (t_b0, t_b1)
    o_b = (o_b0, o_b1)
    semd = (semd0, semd1)
    semt = (semt0, semt1)
    semo = (semo0, semo1)

    pltpu.async_copy(times_hbm.at[pl.ds(base, C3)], t_b[0], semt[0])
    pltpu.async_copy(times_hbm.at[pl.ds(base + C3, C3)], t_b[1], semt[1])

    with jax.named_scope("phaseA"):
        nbase = sid * NPS
        for half in range(NHALF):
            off = nbase + half * APS
            pltpu.sync_copy(rank_hbm.at[pl.ds(off, APS)],
                            dst_b0.at[pl.ds(0, APS)])
            r0 = dst_b0[pl.ds(0, 16)][0]
            r0a = pl.multiple_of(jnp.minimum(
                lax.bitwise_and(r0, jnp.int32(~7)),
                jnp.int32(NN_PAD - WIN)), 8)
            pltpu.sync_copy(mid_hbm.at[pl.ds(r0a, WIN)],
                            dst_b1.at[pl.ds(0, WIN)])

            @plsc.parallel_loop(0, APS // 16, unroll=4)
            def _tblgather(k):
                s = pl.ds(k * 16, 16)
                w = dst_b0[s] - r0a
                o_b0[s] = plsc.bitcast(
                    plsc.load_gather(dst_b1, [w]), jnp.float32)

            pltpu.sync_copy(o_b0.at[pl.ds(0, APS)],
                            tbl_sh.at[pl.ds(off, APS)])

    plsc.subcore_barrier()

    pltpu.async_copy(dst_hbm.at[pl.ds(base, C3)], dst_b[0], semd[0])
    pltpu.async_copy(dst_hbm.at[pl.ds(base + C3, C3)], dst_b[1], semd[1])

    with jax.named_scope("broadcast"):
        pltpu.sync_copy(tbl_sh.at[pl.ds(0, NN)], tbl_v)


    @pl.loop(0, NCH3, step=2)
    def _chunks(j0):
        for b in range(2):
            jj = j0 + b
            pltpu.make_async_copy(
                dst_hbm.at[pl.ds(base + jj * C3, C3)], dst_b[b], semd[b]).wait()
            pltpu.make_async_copy(
                times_hbm.at[pl.ds(base + jj * C3, C3)], t_b[b], semt[b]).wait()

            @pl.when(jj >= 2)
            def _drain_out(_b=b, _jj=jj):
                pltpu.make_async_copy(
                    o_b[_b], out_hbm.at[pl.ds(base + (_jj - 2) * C3, C3)],
                    semo[_b]).wait()

            @plsc.parallel_loop(0, C3 // 16, unroll=10)
            def _ev(k, _d=dst_b[b], _t=t_b[b], _o=o_b[b]):
                s = pl.ds(k * 16, 16)
                idx = _d[s]
                _o[s] = plsc.load_gather(tbl_v, [idx]) - _t[s]
            pltpu.async_copy(
                o_b[b], out_hbm.at[pl.ds(base + jj * C3, C3)], semo[b])

            @pl.when(jj + 2 < NCH3)
            def _prefetch(_b=b, _jj=jj):
                pltpu.async_copy(
                    dst_hbm.at[pl.ds(base + (_jj + 2) * C3, C3)],
                    dst_b[_b], semd[_b])
                pltpu.async_copy(
                    times_hbm.at[pl.ds(base + (_jj + 2) * C3, C3)],
                    t_b[_b], semt[_b])

    pltpu.make_async_copy(
        o_b[0], out_hbm.at[pl.ds(base + (NCH3 - 2) * C3, C3)], semo[0]).wait()
    pltpu.make_async_copy(
        o_b[1], out_hbm.at[pl.ds(base + (NCH3 - 1) * C3, C3)], semo[1]).wait()


def kernel(dst_ids, times, last_update):
    maps, mid = _presence(dst_ids, last_update)
    rank = _rank_tc(maps).reshape(NN_PAD)
    return _main(rank, dst_ids, times, mid)
